# baseline (speedup 1.0000x reference)
import functools

import jax
import jax.numpy as jnp
from jax import lax
from jax.experimental import pallas as pl
from jax.experimental.pallas import tpu as pltpu
from jax.experimental.pallas import tpu_sc as plsc

B, T, D = 4, 2048, 2048
E, TOPK = 8, 2
H = D * 2
N = B * T

TM_R = 1024
NBLK_R = N // TM_R

TM = 512
P = N * TOPK + E * TM
NT = P // TM
BH = 1024
NH = H // BH
BD = 1024
ND = D // BD

NC, NS = 2, 16
NW = NC * NS
CHUNK = N // NW
SUB = 32
NSUB = CHUNK // SUB


def _router_body(x_ref, rw_ref, e0_ref, e1_ref, w0_ref, w1_ref, r0_ref,
                 r1_ref, cnt_ref, ps_ref, counts_ref, psum_ref):
    i = pl.program_id(0)

    @pl.when(i == 0)
    def _():
        counts_ref[...] = jnp.zeros_like(counts_ref)
        psum_ref[...] = jnp.zeros_like(psum_ref)

    xb = x_ref[...]
    rw = rw_ref[...]
    logits = lax.dot_general(xb, rw, (((1,), (1,)), ((), ())),
                             preferred_element_type=jnp.float32)
    m = jnp.max(logits, axis=-1, keepdims=True)
    unn = jnp.exp(logits - m)
    probs = unn / jnp.sum(unn, axis=-1, keepdims=True)

    iota = lax.broadcasted_iota(jnp.int32, (TM_R, E), 1)
    m0 = jnp.max(probs, axis=-1, keepdims=True)
    a0 = jnp.min(jnp.where(probs == m0, iota, E), axis=-1, keepdims=True)
    oh0 = (iota == a0).astype(jnp.float32)
    pm = jnp.where(iota == a0, -jnp.inf, probs)
    m1 = jnp.max(pm, axis=-1, keepdims=True)
    a1 = jnp.min(jnp.where(pm == m1, iota, E), axis=-1, keepdims=True)
    oh1 = (iota == a1).astype(jnp.float32)

    s = m0 + m1
    w0 = m0 / s
    w1v = m1 / s

    contrib = oh0 + oh1
    ri = lax.broadcasted_iota(jnp.int32, (TM_R, TM_R), 0)
    ci = lax.broadcasted_iota(jnp.int32, (TM_R, TM_R), 1)
    tri = (ri > ci).astype(jnp.float32)
    excl = lax.dot_general(tri, contrib, (((1,), (0,)), ((), ())),
                           preferred_element_type=jnp.float32)
    base = counts_ref[...]
    tot = excl + base
    r0 = jnp.sum(tot * oh0, axis=-1)
    r1 = jnp.sum(tot * oh1, axis=-1)
    counts_ref[...] = base + jnp.sum(contrib, axis=0, keepdims=True)
    psum_ref[...] = psum_ref[...] + jnp.sum(probs, axis=0, keepdims=True)

    e0_ref[...] = a0[:, 0].astype(jnp.int32).reshape(1, 1, TM_R)
    e1_ref[...] = a1[:, 0].astype(jnp.int32).reshape(1, 1, TM_R)
    w0_ref[...] = w0[:, 0].reshape(1, 1, TM_R)
    w1_ref[...] = w1v[:, 0].reshape(1, 1, TM_R)
    r0_ref[...] = r0.astype(jnp.int32).reshape(1, 1, TM_R)
    r1_ref[...] = r1.astype(jnp.int32).reshape(1, 1, TM_R)

    @pl.when(i == NBLK_R - 1)
    def _():
        cnt_ref[...] = counts_ref[...].astype(jnp.int32)
        ps_ref[...] = psum_ref[...]


def _router(x2d, router_w):
    blk3 = lambda i: (i, 0, 0)
    out3_i = jax.ShapeDtypeStruct((NBLK_R, 1, TM_R), jnp.int32)
    out3_f = jax.ShapeDtypeStruct((NBLK_R, 1, TM_R), jnp.float32)
    return pl.pallas_call(
        _router_body,
        grid=(NBLK_R,),
        in_specs=[
            pl.BlockSpec((TM_R, D), lambda i: (i, 0)),
            pl.BlockSpec((E, D), lambda i: (0, 0)),
        ],
        out_specs=[
            pl.BlockSpec((1, 1, TM_R), blk3),
            pl.BlockSpec((1, 1, TM_R), blk3),
            pl.BlockSpec((1, 1, TM_R), blk3),
            pl.BlockSpec((1, 1, TM_R), blk3),
            pl.BlockSpec((1, 1, TM_R), blk3),
            pl.BlockSpec((1, 1, TM_R), blk3),
            pl.BlockSpec((1, E), lambda i: (0, 0)),
            pl.BlockSpec((1, E), lambda i: (0, 0)),
        ],
        out_shape=[out3_i, out3_i, out3_f, out3_f, out3_i, out3_i,
                   jax.ShapeDtypeStruct((1, E), jnp.int32),
                   jax.ShapeDtypeStruct((1, E), jnp.float32)],
        scratch_shapes=[pltpu.VMEM((1, E), jnp.float32),
                        pltpu.VMEM((1, E), jnp.float32)],
    )(x2d, router_w)


def _sc_scatter_body(x_hbm, e0_hbm, e1_hbm, r0_hbm, r1_hbm, base_hbm,
                     xg_hbm, s0_hbm, s1_hbm,
                     base_v, ev, rv, s0v, s1v, idx0, idx1, xrows):
    wid = lax.axis_index("s") * NC + lax.axis_index("c")
    tb = wid * CHUNK
    pltpu.sync_copy(base_hbm, base_v.at[pl.ds(0, E)])
    bvec = base_v[...]
    base_sc = [bvec[e] for e in range(E)]
    for k in range(2):
        e_hbm = e0_hbm if k == 0 else e1_hbm
        r_hbm = r0_hbm if k == 0 else r1_hbm
        s_hbm = s0_hbm if k == 0 else s1_hbm
        sv = s0v if k == 0 else s1v
        pltpu.sync_copy(e_hbm.at[pl.ds(tb, CHUNK)], ev)
        pltpu.sync_copy(r_hbm.at[pl.ds(tb, CHUNK)], rv)
        for i in range(CHUNK // 16):
            e16 = ev[pl.ds(i * 16, 16)]
            r16 = rv[pl.ds(i * 16, 16)]
            b16 = jnp.full((16,), base_sc[0], jnp.int32)
            for e in range(1, E):
                b16 = jnp.where(e16 == e, base_sc[e], b16)
            sv[pl.ds(i * 16, 16)] = b16 + r16
        pltpu.sync_copy(sv, s_hbm.at[pl.ds(tb, CHUNK)])
    for j in range(NSUB):
        pltpu.sync_copy(x_hbm.at[pl.ds(tb + j * SUB, SUB)], xrows)
        for i in range(SUB // 16):
            idx0[pl.ds(i * 16, 16)] = s0v[pl.ds(j * SUB + i * 16, 16)]
            idx1[pl.ds(i * 16, 16)] = s1v[pl.ds(j * SUB + i * 16, 16)]
        pltpu.sync_copy(xrows, xg_hbm.at[idx0])
        pltpu.sync_copy(xrows, xg_hbm.at[idx1])


def _new_weight_block(te_ref):
    m = pl.program_id(1)
    prev = te_ref[jnp.maximum(m - 1, 0)]
    return (m == 0) | (te_ref[m] != prev)


def _mm1_body(te_ref, xg_ref, w1_ref, w2_ref, hall_ref, w1c_ref, w2c_ref):
    @pl.when(_new_weight_block(te_ref))
    def _():
        w1c_ref[...] = w1_ref[0].astype(jnp.bfloat16)
        w2c_ref[...] = w2_ref[0].astype(jnp.bfloat16)

    xb = xg_ref[...].astype(jnp.bfloat16)
    t1 = jnp.dot(xb, w1c_ref[...], preferred_element_type=jnp.float32)
    t2 = jnp.dot(xb, w2c_ref[...], preferred_element_type=jnp.float32)
    hall_ref[...] = (t1 * jax.nn.sigmoid(t1) * t2).astype(jnp.bfloat16)


def _mm2_body(te_ref, hall_ref, w3_ref, yg_ref, w3c_ref):
    @pl.when(_new_weight_block(te_ref))
    def _():
        w3c_ref[...] = w3_ref[0].astype(jnp.bfloat16)

    hb = hall_ref[...]
    yg_ref[...] = jnp.dot(hb, w3c_ref[...], preferred_element_type=jnp.float32)


def _mm(te, xg, w1, w2, w3):
    gs1 = pltpu.PrefetchScalarGridSpec(
        num_scalar_prefetch=1,
        grid=(NH, NT),
        in_specs=[
            pl.BlockSpec((TM, D), lambda h, m, te: (m, 0)),
            pl.BlockSpec((1, D, BH), lambda h, m, te: (te[m], 0, h)),
            pl.BlockSpec((1, D, BH), lambda h, m, te: (te[m], 0, h)),
        ],
        out_specs=pl.BlockSpec((TM, BH), lambda h, m, te: (m, h)),
        scratch_shapes=[pltpu.VMEM((D, BH), jnp.bfloat16),
                        pltpu.VMEM((D, BH), jnp.bfloat16)],
    )
    hall = pl.pallas_call(
        _mm1_body,
        grid_spec=gs1,
        out_shape=jax.ShapeDtypeStruct((P, H), jnp.bfloat16),
    )(te, xg, w1, w2)
    gs2 = pltpu.PrefetchScalarGridSpec(
        num_scalar_prefetch=1,
        grid=(ND, NT),
        in_specs=[
            pl.BlockSpec((TM, H), lambda d, m, te: (m, 0)),
            pl.BlockSpec((1, H, BD), lambda d, m, te: (te[m], 0, d)),
        ],
        out_specs=pl.BlockSpec((TM, BD), lambda d, m, te: (m, d)),
        scratch_shapes=[pltpu.VMEM((H, BD), jnp.bfloat16)],
    )
    return pl.pallas_call(
        _mm2_body,
        grid_spec=gs2,
        out_shape=jax.ShapeDtypeStruct((P, D), jnp.float32),
    )(te, hall, w3)


def _sc_gather_body(yg_hbm, s0_hbm, s1_hbm, y0_hbm, y1_hbm, sv, idx, rows):
    wid = lax.axis_index("s") * NC + lax.axis_index("c")
    tb = wid * CHUNK
    for k in range(2):
        s_hbm = s0_hbm if k == 0 else s1_hbm
        o_hbm = y0_hbm if k == 0 else y1_hbm
        pltpu.sync_copy(s_hbm.at[pl.ds(tb, CHUNK)], sv)
        for j in range(NSUB):
            for i in range(SUB // 16):
                idx[pl.ds(i * 16, 16)] = sv[pl.ds(j * SUB + i * 16, 16)]
            pltpu.sync_copy(yg_hbm.at[idx], rows)
            pltpu.sync_copy(rows, o_hbm.at[pl.ds(tb + j * SUB, SUB)])


@functools.cache
def _sc_kernels():
    mesh = plsc.VectorSubcoreMesh(core_axis_name="c", subcore_axis_name="s")
    scatter = functools.partial(
        pl.kernel, mesh=mesh,
        out_type=[jax.ShapeDtypeStruct((P, D), jnp.float32),
                  jax.ShapeDtypeStruct((N,), jnp.int32),
                  jax.ShapeDtypeStruct((N,), jnp.int32)],
        scratch_types=[pltpu.VMEM((16,), jnp.int32),
                       pltpu.VMEM((CHUNK,), jnp.int32),
                       pltpu.VMEM((CHUNK,), jnp.int32),
                       pltpu.VMEM((CHUNK,), jnp.int32),
                       pltpu.VMEM((CHUNK,), jnp.int32),
                       pltpu.VMEM((SUB,), jnp.int32),
                       pltpu.VMEM((SUB,), jnp.int32),
                       pltpu.VMEM((SUB, D), jnp.float32)],
    )(_sc_scatter_body)
    gather = functools.partial(
        pl.kernel, mesh=mesh,
        out_type=[jax.ShapeDtypeStruct((N, D), jnp.float32),
                  jax.ShapeDtypeStruct((N, D), jnp.float32)],
        scratch_types=[pltpu.VMEM((CHUNK,), jnp.int32),
                       pltpu.VMEM((SUB,), jnp.int32),
                       pltpu.VMEM((SUB, D), jnp.float32)],
    )(_sc_gather_body)
    return scatter, gather


TMC = 512


def _combine_body(y0_ref, y1_ref, w0_ref, w1_ref, o_ref):
    o_ref[...] = w0_ref[...] * y0_ref[...] + w1_ref[...] * y1_ref[...]


def _combine(y0, y1, w0c, w1c):
    return pl.pallas_call(
        _combine_body,
        grid=(N // TMC,),
        in_specs=[
            pl.BlockSpec((TMC, D), lambda i: (i, 0)),
            pl.BlockSpec((TMC, D), lambda i: (i, 0)),
            pl.BlockSpec((TMC, 1), lambda i: (i, 0)),
            pl.BlockSpec((TMC, 1), lambda i: (i, 0)),
        ],
        out_specs=pl.BlockSpec((TMC, D), lambda i: (i, 0)),
        out_shape=jax.ShapeDtypeStruct((N, D), jnp.float32),
    )(y0, y1, w0c, w1c)


def kernel(x, router_w, w1, w2, w3):
    x2d = x.reshape(N, D)
    (e0b, e1b, w0b, w1b, r0b, r1b, cnt, ps) = _router(x2d, router_w)
    e0 = e0b.reshape(N)
    e1 = e1b.reshape(N)
    r0 = r0b.reshape(N)
    r1 = r1b.reshape(N)

    counts = cnt.reshape(E)
    padded = ((counts + (TM - 1)) // TM) * TM
    ends = jnp.cumsum(padded)
    base = (ends - padded).astype(jnp.int32)
    tile_start = jnp.arange(NT, dtype=jnp.int32) * TM
    te = jnp.minimum(
        jnp.sum((tile_start[:, None] >= ends[None, :]).astype(jnp.int32),
                axis=1), E - 1).astype(jnp.int32)

    sc_scatter, sc_gather = _sc_kernels()
    xg, s0, s1 = sc_scatter(x2d, e0, e1, r0, r1, base)
    yg = _mm(te, xg, w1, w2, w3)
    y0, y1 = sc_gather(yg, s0, s1)
    out = _combine(y0, y1, w0b.reshape(N, 1), w1b.reshape(N, 1))

    fraction = ps.reshape(E) / jnp.float32(N)
    target = jnp.ones((E,), jnp.float32) / E
    aux = jnp.sum(fraction * jnp.log(target)) * E
    return out.reshape(B, T, D), aux


# speedup vs baseline: 1.0597x; 1.0597x over previous
import functools

import jax
import jax.numpy as jnp
from jax import lax
from jax.experimental import pallas as pl
from jax.experimental.pallas import tpu as pltpu
from jax.experimental.pallas import tpu_sc as plsc

B, T, D = 4, 2048, 2048
E, TOPK = 8, 2
H = D * 2
N = B * T

TM_R = 1024
NBLK_R = N // TM_R

TM = 512
P = N * TOPK + E * TM
NT = P // TM
BH = 1024
NH = H // BH
BD = 1024
ND = D // BD

NC, NS = 2, 16
NW = NC * NS
CHUNK = N // NW
SUB = 32
NSUB = CHUNK // SUB


def _router_body(x_ref, rw_ref, e0_ref, e1_ref, w0_ref, w1_ref, r0_ref,
                 r1_ref, cnt_ref, ps_ref, counts_ref, psum_ref):
    i = pl.program_id(0)

    @pl.when(i == 0)
    def _():
        counts_ref[...] = jnp.zeros_like(counts_ref)
        psum_ref[...] = jnp.zeros_like(psum_ref)

    xb = x_ref[...]
    rw = rw_ref[...]
    logits = lax.dot_general(xb, rw, (((1,), (1,)), ((), ())),
                             preferred_element_type=jnp.float32)
    m = jnp.max(logits, axis=-1, keepdims=True)
    unn = jnp.exp(logits - m)
    probs = unn / jnp.sum(unn, axis=-1, keepdims=True)

    iota = lax.broadcasted_iota(jnp.int32, (TM_R, E), 1)
    m0 = jnp.max(probs, axis=-1, keepdims=True)
    a0 = jnp.min(jnp.where(probs == m0, iota, E), axis=-1, keepdims=True)
    oh0 = (iota == a0).astype(jnp.float32)
    pm = jnp.where(iota == a0, -jnp.inf, probs)
    m1 = jnp.max(pm, axis=-1, keepdims=True)
    a1 = jnp.min(jnp.where(pm == m1, iota, E), axis=-1, keepdims=True)
    oh1 = (iota == a1).astype(jnp.float32)

    s = m0 + m1
    w0 = m0 / s
    w1v = m1 / s

    contrib = oh0 + oh1
    ri = lax.broadcasted_iota(jnp.int32, (TM_R, TM_R), 0)
    ci = lax.broadcasted_iota(jnp.int32, (TM_R, TM_R), 1)
    tri = (ri > ci).astype(jnp.float32)
    excl = lax.dot_general(tri, contrib, (((1,), (0,)), ((), ())),
                           preferred_element_type=jnp.float32)
    base = counts_ref[...]
    tot = excl + base
    r0 = jnp.sum(tot * oh0, axis=-1)
    r1 = jnp.sum(tot * oh1, axis=-1)
    counts_ref[...] = base + jnp.sum(contrib, axis=0, keepdims=True)
    psum_ref[...] = psum_ref[...] + jnp.sum(probs, axis=0, keepdims=True)

    e0_ref[...] = a0[:, 0].astype(jnp.int32).reshape(1, 1, TM_R)
    e1_ref[...] = a1[:, 0].astype(jnp.int32).reshape(1, 1, TM_R)
    w0_ref[...] = w0[:, 0].reshape(1, 1, TM_R)
    w1_ref[...] = w1v[:, 0].reshape(1, 1, TM_R)
    r0_ref[...] = r0.astype(jnp.int32).reshape(1, 1, TM_R)
    r1_ref[...] = r1.astype(jnp.int32).reshape(1, 1, TM_R)

    @pl.when(i == NBLK_R - 1)
    def _():
        cnt_ref[...] = counts_ref[...].astype(jnp.int32)
        ps_ref[...] = psum_ref[...]


def _router(x2d, router_w):
    blk3 = lambda i: (i, 0, 0)
    out3_i = jax.ShapeDtypeStruct((NBLK_R, 1, TM_R), jnp.int32)
    out3_f = jax.ShapeDtypeStruct((NBLK_R, 1, TM_R), jnp.float32)
    return pl.pallas_call(
        _router_body,
        grid=(NBLK_R,),
        in_specs=[
            pl.BlockSpec((TM_R, D), lambda i: (i, 0)),
            pl.BlockSpec((E, D), lambda i: (0, 0)),
        ],
        out_specs=[
            pl.BlockSpec((1, 1, TM_R), blk3),
            pl.BlockSpec((1, 1, TM_R), blk3),
            pl.BlockSpec((1, 1, TM_R), blk3),
            pl.BlockSpec((1, 1, TM_R), blk3),
            pl.BlockSpec((1, 1, TM_R), blk3),
            pl.BlockSpec((1, 1, TM_R), blk3),
            pl.BlockSpec((1, E), lambda i: (0, 0)),
            pl.BlockSpec((1, E), lambda i: (0, 0)),
        ],
        out_shape=[out3_i, out3_i, out3_f, out3_f, out3_i, out3_i,
                   jax.ShapeDtypeStruct((1, E), jnp.int32),
                   jax.ShapeDtypeStruct((1, E), jnp.float32)],
        scratch_shapes=[pltpu.VMEM((1, E), jnp.float32),
                        pltpu.VMEM((1, E), jnp.float32)],
    )(x2d, router_w)


def _sc_scatter_body(x_hbm, e0_hbm, e1_hbm, r0_hbm, r1_hbm, base_hbm,
                     xg_hbm, s0_hbm, s1_hbm,
                     base_v, ev, rv, s0v, s1v, idx0, idx1, xrows):
    wid = lax.axis_index("s") * NC + lax.axis_index("c")
    tb = wid * CHUNK
    pltpu.sync_copy(base_hbm, base_v.at[pl.ds(0, E)])
    bvec = base_v[...]
    base_sc = [bvec[e] for e in range(E)]
    for k in range(2):
        e_hbm = e0_hbm if k == 0 else e1_hbm
        r_hbm = r0_hbm if k == 0 else r1_hbm
        s_hbm = s0_hbm if k == 0 else s1_hbm
        sv = s0v if k == 0 else s1v
        pltpu.sync_copy(e_hbm.at[pl.ds(tb, CHUNK)], ev)
        pltpu.sync_copy(r_hbm.at[pl.ds(tb, CHUNK)], rv)
        for i in range(CHUNK // 16):
            e16 = ev[pl.ds(i * 16, 16)]
            r16 = rv[pl.ds(i * 16, 16)]
            b16 = jnp.full((16,), base_sc[0], jnp.int32)
            for e in range(1, E):
                b16 = jnp.where(e16 == e, base_sc[e], b16)
            sv[pl.ds(i * 16, 16)] = b16 + r16
        pltpu.sync_copy(sv, s_hbm.at[pl.ds(tb, CHUNK)])
    for j in range(NSUB):
        pltpu.sync_copy(x_hbm.at[pl.ds(tb + j * SUB, SUB)], xrows)
        for i in range(SUB // 16):
            idx0[pl.ds(i * 16, 16)] = s0v[pl.ds(j * SUB + i * 16, 16)]
            idx1[pl.ds(i * 16, 16)] = s1v[pl.ds(j * SUB + i * 16, 16)]
        pltpu.sync_copy(xrows, xg_hbm.at[idx0])
        pltpu.sync_copy(xrows, xg_hbm.at[idx1])


def _mm1_body(te_ref, nt_ref, xg_ref, w1_ref, w2_ref, hall_ref):
    m = pl.program_id(1)

    @pl.when(m < nt_ref[0])
    def _():
        xb = xg_ref[...].astype(jnp.bfloat16)
        t1 = jnp.dot(xb, w1_ref[0].astype(jnp.bfloat16),
                     preferred_element_type=jnp.float32)
        t2 = jnp.dot(xb, w2_ref[0].astype(jnp.bfloat16),
                     preferred_element_type=jnp.float32)
        hall_ref[...] = (t1 * jax.nn.sigmoid(t1) * t2).astype(jnp.bfloat16)


def _mm2_body(te_ref, nt_ref, hall_ref, w3_ref, yg_ref):
    m = pl.program_id(1)

    @pl.when(m < nt_ref[0])
    def _():
        hb = hall_ref[...]
        yg_ref[...] = jnp.dot(hb, w3_ref[0].astype(jnp.bfloat16),
                              preferred_element_type=jnp.float32)


def _mm(te, nt_used, xg, w1, w2, w3):
    gs1 = pltpu.PrefetchScalarGridSpec(
        num_scalar_prefetch=2,
        grid=(NH, NT),
        in_specs=[
            pl.BlockSpec((TM, D), lambda h, m, te, nt: (m, 0)),
            pl.BlockSpec((1, D, BH), lambda h, m, te, nt: (te[m], 0, h)),
            pl.BlockSpec((1, D, BH), lambda h, m, te, nt: (te[m], 0, h)),
        ],
        out_specs=pl.BlockSpec((TM, BH), lambda h, m, te, nt: (m, h)),
    )
    hall = pl.pallas_call(
        _mm1_body,
        grid_spec=gs1,
        out_shape=jax.ShapeDtypeStruct((P, H), jnp.bfloat16),
    )(te, nt_used, xg, w1, w2)
    gs2 = pltpu.PrefetchScalarGridSpec(
        num_scalar_prefetch=2,
        grid=(ND, NT),
        in_specs=[
            pl.BlockSpec((TM, H), lambda d, m, te, nt: (m, 0)),
            pl.BlockSpec((1, H, BD), lambda d, m, te, nt: (te[m], 0, d)),
        ],
        out_specs=pl.BlockSpec((TM, BD), lambda d, m, te, nt: (m, d)),
    )
    return pl.pallas_call(
        _mm2_body,
        grid_spec=gs2,
        out_shape=jax.ShapeDtypeStruct((P, D), jnp.float32),
    )(te, nt_used, hall, w3)


def _sc_gather_body(yg_hbm, s0_hbm, s1_hbm, y0_hbm, y1_hbm, sv, idx, rows):
    wid = lax.axis_index("s") * NC + lax.axis_index("c")
    tb = wid * CHUNK
    for k in range(2):
        s_hbm = s0_hbm if k == 0 else s1_hbm
        o_hbm = y0_hbm if k == 0 else y1_hbm
        pltpu.sync_copy(s_hbm.at[pl.ds(tb, CHUNK)], sv)
        for j in range(NSUB):
            for i in range(SUB // 16):
                idx[pl.ds(i * 16, 16)] = sv[pl.ds(j * SUB + i * 16, 16)]
            pltpu.sync_copy(yg_hbm.at[idx], rows)
            pltpu.sync_copy(rows, o_hbm.at[pl.ds(tb + j * SUB, SUB)])


@functools.cache
def _sc_kernels():
    mesh = plsc.VectorSubcoreMesh(core_axis_name="c", subcore_axis_name="s")
    scatter = functools.partial(
        pl.kernel, mesh=mesh,
        out_type=[jax.ShapeDtypeStruct((P, D), jnp.float32),
                  jax.ShapeDtypeStruct((N,), jnp.int32),
                  jax.ShapeDtypeStruct((N,), jnp.int32)],
        scratch_types=[pltpu.VMEM((16,), jnp.int32),
                       pltpu.VMEM((CHUNK,), jnp.int32),
                       pltpu.VMEM((CHUNK,), jnp.int32),
                       pltpu.VMEM((CHUNK,), jnp.int32),
                       pltpu.VMEM((CHUNK,), jnp.int32),
                       pltpu.VMEM((SUB,), jnp.int32),
                       pltpu.VMEM((SUB,), jnp.int32),
                       pltpu.VMEM((SUB, D), jnp.float32)],
    )(_sc_scatter_body)
    gather = functools.partial(
        pl.kernel, mesh=mesh,
        out_type=[jax.ShapeDtypeStruct((N, D), jnp.float32),
                  jax.ShapeDtypeStruct((N, D), jnp.float32)],
        scratch_types=[pltpu.VMEM((CHUNK,), jnp.int32),
                       pltpu.VMEM((SUB,), jnp.int32),
                       pltpu.VMEM((SUB, D), jnp.float32)],
    )(_sc_gather_body)
    return scatter, gather


TMC = 512


def _combine_body(y0_ref, y1_ref, w0_ref, w1_ref, o_ref):
    o_ref[...] = w0_ref[...] * y0_ref[...] + w1_ref[...] * y1_ref[...]


def _combine(y0, y1, w0c, w1c):
    return pl.pallas_call(
        _combine_body,
        grid=(N // TMC,),
        in_specs=[
            pl.BlockSpec((TMC, D), lambda i: (i, 0)),
            pl.BlockSpec((TMC, D), lambda i: (i, 0)),
            pl.BlockSpec((TMC, 1), lambda i: (i, 0)),
            pl.BlockSpec((TMC, 1), lambda i: (i, 0)),
        ],
        out_specs=pl.BlockSpec((TMC, D), lambda i: (i, 0)),
        out_shape=jax.ShapeDtypeStruct((N, D), jnp.float32),
    )(y0, y1, w0c, w1c)


def kernel(x, router_w, w1, w2, w3):
    x2d = x.reshape(N, D)
    (e0b, e1b, w0b, w1b, r0b, r1b, cnt, ps) = _router(x2d, router_w)
    e0 = e0b.reshape(N)
    e1 = e1b.reshape(N)
    r0 = r0b.reshape(N)
    r1 = r1b.reshape(N)

    counts = cnt.reshape(E)
    padded = ((counts + (TM - 1)) // TM) * TM
    ends = jnp.cumsum(padded)
    base = (ends - padded).astype(jnp.int32)
    tile_start = jnp.arange(NT, dtype=jnp.int32) * TM
    te = jnp.minimum(
        jnp.sum((tile_start[:, None] >= ends[None, :]).astype(jnp.int32),
                axis=1), E - 1).astype(jnp.int32)

    nt_used = ((ends[E - 1] + (TM - 1)) // TM).reshape(1).astype(jnp.int32)

    sc_scatter, sc_gather = _sc_kernels()
    xg, s0, s1 = sc_scatter(x2d, e0, e1, r0, r1, base)
    yg = _mm(te, nt_used, xg, w1, w2, w3)
    y0, y1 = sc_gather(yg, s0, s1)
    out = _combine(y0, y1, w0b.reshape(N, 1), w1b.reshape(N, 1))

    fraction = ps.reshape(E) / jnp.float32(N)
    target = jnp.ones((E,), jnp.float32) / E
    aux = jnp.sum(fraction * jnp.log(target)) * E
    return out.reshape(B, T, D), aux


# speedup vs baseline: 1.0959x; 1.0342x over previous
import functools

import jax
import jax.numpy as jnp
from jax import lax
from jax.experimental import pallas as pl
from jax.experimental.pallas import tpu as pltpu
from jax.experimental.pallas import tpu_sc as plsc

B, T, D = 4, 2048, 2048
E, TOPK = 8, 2
H = D * 2
N = B * T

TM_R = 1024
NBLK_R = N // TM_R

TM = 512
P = N * TOPK + E * TM
NT = P // TM
BH = 1024
NH = H // BH
BD = 1024
ND = D // BD

NC, NS = 2, 16
NW = NC * NS
CHUNK = N // NW
SUB = 32
NSUB = CHUNK // SUB


def _router_body(x_ref, rw_ref, e0_ref, e1_ref, w0_ref, w1_ref, r0_ref,
                 r1_ref, cnt_ref, ps_ref, counts_ref, psum_ref):
    i = pl.program_id(0)

    @pl.when(i == 0)
    def _():
        counts_ref[...] = jnp.zeros_like(counts_ref)
        psum_ref[...] = jnp.zeros_like(psum_ref)

    xb = x_ref[...]
    rw = rw_ref[...]
    logits = lax.dot_general(xb, rw, (((1,), (1,)), ((), ())),
                             preferred_element_type=jnp.float32)
    m = jnp.max(logits, axis=-1, keepdims=True)
    unn = jnp.exp(logits - m)
    probs = unn / jnp.sum(unn, axis=-1, keepdims=True)

    iota = lax.broadcasted_iota(jnp.int32, (TM_R, E), 1)
    m0 = jnp.max(probs, axis=-1, keepdims=True)
    a0 = jnp.min(jnp.where(probs == m0, iota, E), axis=-1, keepdims=True)
    oh0 = (iota == a0).astype(jnp.float32)
    pm = jnp.where(iota == a0, -jnp.inf, probs)
    m1 = jnp.max(pm, axis=-1, keepdims=True)
    a1 = jnp.min(jnp.where(pm == m1, iota, E), axis=-1, keepdims=True)
    oh1 = (iota == a1).astype(jnp.float32)

    s = m0 + m1
    w0 = m0 / s
    w1v = m1 / s

    contrib = oh0 + oh1
    ri = lax.broadcasted_iota(jnp.int32, (TM_R, TM_R), 0)
    ci = lax.broadcasted_iota(jnp.int32, (TM_R, TM_R), 1)
    tri = (ri > ci).astype(jnp.float32)
    excl = lax.dot_general(tri, contrib, (((1,), (0,)), ((), ())),
                           preferred_element_type=jnp.float32)
    base = counts_ref[...]
    tot = excl + base
    r0 = jnp.sum(tot * oh0, axis=-1)
    r1 = jnp.sum(tot * oh1, axis=-1)
    counts_ref[...] = base + jnp.sum(contrib, axis=0, keepdims=True)
    psum_ref[...] = psum_ref[...] + jnp.sum(probs, axis=0, keepdims=True)

    e0_ref[...] = a0[:, 0].astype(jnp.int32).reshape(1, 1, TM_R)
    e1_ref[...] = a1[:, 0].astype(jnp.int32).reshape(1, 1, TM_R)
    w0_ref[...] = w0[:, 0].reshape(1, 1, TM_R)
    w1_ref[...] = w1v[:, 0].reshape(1, 1, TM_R)
    r0_ref[...] = r0.astype(jnp.int32).reshape(1, 1, TM_R)
    r1_ref[...] = r1.astype(jnp.int32).reshape(1, 1, TM_R)

    @pl.when(i == NBLK_R - 1)
    def _():
        cnt_ref[...] = counts_ref[...].astype(jnp.int32)
        ps_ref[...] = psum_ref[...]


def _router(x2d, router_w):
    blk3 = lambda i: (i, 0, 0)
    out3_i = jax.ShapeDtypeStruct((NBLK_R, 1, TM_R), jnp.int32)
    out3_f = jax.ShapeDtypeStruct((NBLK_R, 1, TM_R), jnp.float32)
    return pl.pallas_call(
        _router_body,
        grid=(NBLK_R,),
        in_specs=[
            pl.BlockSpec((TM_R, D), lambda i: (i, 0)),
            pl.BlockSpec((E, D), lambda i: (0, 0)),
        ],
        out_specs=[
            pl.BlockSpec((1, 1, TM_R), blk3),
            pl.BlockSpec((1, 1, TM_R), blk3),
            pl.BlockSpec((1, 1, TM_R), blk3),
            pl.BlockSpec((1, 1, TM_R), blk3),
            pl.BlockSpec((1, 1, TM_R), blk3),
            pl.BlockSpec((1, 1, TM_R), blk3),
            pl.BlockSpec((1, E), lambda i: (0, 0)),
            pl.BlockSpec((1, E), lambda i: (0, 0)),
        ],
        out_shape=[out3_i, out3_i, out3_f, out3_f, out3_i, out3_i,
                   jax.ShapeDtypeStruct((1, E), jnp.int32),
                   jax.ShapeDtypeStruct((1, E), jnp.float32)],
        scratch_shapes=[pltpu.VMEM((1, E), jnp.float32),
                        pltpu.VMEM((1, E), jnp.float32)],
    )(x2d, router_w)


def _sc_scatter_body(x_hbm, e0_hbm, e1_hbm, r0_hbm, r1_hbm, w0_hbm, w1_hbm,
                     base_hbm, xg_hbm, s0_hbm, s1_hbm, ws_hbm,
                     base_v, ev, rv, s0v, s1v, wv, idx0, idx1, xrows):
    wid = lax.axis_index("s") * NC + lax.axis_index("c")
    tb = wid * CHUNK
    pltpu.sync_copy(base_hbm, base_v.at[pl.ds(0, E)])
    bvec = base_v[...]
    base_sc = [bvec[e] for e in range(E)]
    for k in range(2):
        e_hbm = e0_hbm if k == 0 else e1_hbm
        r_hbm = r0_hbm if k == 0 else r1_hbm
        s_hbm = s0_hbm if k == 0 else s1_hbm
        sv = s0v if k == 0 else s1v
        pltpu.sync_copy(e_hbm.at[pl.ds(tb, CHUNK)], ev)
        pltpu.sync_copy(r_hbm.at[pl.ds(tb, CHUNK)], rv)
        for i in range(CHUNK // 16):
            e16 = ev[pl.ds(i * 16, 16)]
            r16 = rv[pl.ds(i * 16, 16)]
            b16 = jnp.full((16,), base_sc[0], jnp.int32)
            for e in range(1, E):
                b16 = jnp.where(e16 == e, base_sc[e], b16)
            sv[pl.ds(i * 16, 16)] = b16 + r16
        pltpu.sync_copy(sv, s_hbm.at[pl.ds(tb, CHUNK)])
    for j in range(NSUB):
        pltpu.sync_copy(x_hbm.at[pl.ds(tb + j * SUB, SUB)], xrows)
        for i in range(SUB // 16):
            idx0[pl.ds(i * 16, 16)] = s0v[pl.ds(j * SUB + i * 16, 16)]
            idx1[pl.ds(i * 16, 16)] = s1v[pl.ds(j * SUB + i * 16, 16)]
        pltpu.sync_copy(xrows, xg_hbm.at[idx0])
        pltpu.sync_copy(xrows, xg_hbm.at[idx1])
        pltpu.sync_copy(w0_hbm.at[pl.ds(tb + j * SUB, SUB)], wv)
        pltpu.sync_copy(wv, ws_hbm.at[idx0])
        pltpu.sync_copy(w1_hbm.at[pl.ds(tb + j * SUB, SUB)], wv)
        pltpu.sync_copy(wv, ws_hbm.at[idx1])


def _mm1_body(te_ref, nt_ref, xg_ref, w1_ref, w2_ref, hall_ref):
    m = pl.program_id(1)

    @pl.when(m < nt_ref[0])
    def _():
        xb = xg_ref[...].astype(jnp.bfloat16)
        t1 = jnp.dot(xb, w1_ref[0].astype(jnp.bfloat16),
                     preferred_element_type=jnp.float32)
        t2 = jnp.dot(xb, w2_ref[0].astype(jnp.bfloat16),
                     preferred_element_type=jnp.float32)
        hall_ref[...] = (t1 * jax.nn.sigmoid(t1) * t2).astype(jnp.bfloat16)


def _mm2_body(te_ref, nt_ref, hall_ref, w3_ref, ws_ref, yg_ref):
    m = pl.program_id(1)

    @pl.when(m < nt_ref[0])
    def _():
        hb = hall_ref[...]
        yb = jnp.dot(hb, w3_ref[0].astype(jnp.bfloat16),
                     preferred_element_type=jnp.float32)
        yg_ref[...] = yb * ws_ref[...]


def _mm(te, nt_used, xg, w1, w2, w3, ws):
    gs1 = pltpu.PrefetchScalarGridSpec(
        num_scalar_prefetch=2,
        grid=(NH, NT),
        in_specs=[
            pl.BlockSpec((TM, D), lambda h, m, te, nt: (m, 0)),
            pl.BlockSpec((1, D, BH), lambda h, m, te, nt: (te[m], 0, h)),
            pl.BlockSpec((1, D, BH), lambda h, m, te, nt: (te[m], 0, h)),
        ],
        out_specs=pl.BlockSpec((TM, BH), lambda h, m, te, nt: (m, h)),
    )
    hall = pl.pallas_call(
        _mm1_body,
        grid_spec=gs1,
        out_shape=jax.ShapeDtypeStruct((P, H), jnp.bfloat16),
    )(te, nt_used, xg, w1, w2)
    gs2 = pltpu.PrefetchScalarGridSpec(
        num_scalar_prefetch=2,
        grid=(ND, NT),
        in_specs=[
            pl.BlockSpec((TM, H), lambda d, m, te, nt: (m, 0)),
            pl.BlockSpec((1, H, BD), lambda d, m, te, nt: (te[m], 0, d)),
            pl.BlockSpec((TM, 1), lambda d, m, te, nt: (m, 0)),
        ],
        out_specs=pl.BlockSpec((TM, BD), lambda d, m, te, nt: (m, d)),
    )
    return pl.pallas_call(
        _mm2_body,
        grid_spec=gs2,
        out_shape=jax.ShapeDtypeStruct((P, D), jnp.float32),
    )(te, nt_used, hall, w3, ws)


def _sc_gather_body(yg_hbm, s0_hbm, s1_hbm, out_hbm,
                    sv0, sv1, idx0, idx1, rows):
    wid = lax.axis_index("s") * NC + lax.axis_index("c")
    tb = wid * CHUNK
    pltpu.sync_copy(s0_hbm.at[pl.ds(tb, CHUNK)], sv0)
    pltpu.sync_copy(s1_hbm.at[pl.ds(tb, CHUNK)], sv1)
    for j in range(NSUB):
        for i in range(SUB // 16):
            idx0[pl.ds(i * 16, 16)] = sv0[pl.ds(j * SUB + i * 16, 16)]
            idx1[pl.ds(i * 16, 16)] = sv1[pl.ds(j * SUB + i * 16, 16)]
        pltpu.sync_copy(yg_hbm.at[idx0], rows)
        pltpu.sync_copy(yg_hbm.at[idx1], rows, add=True)
        pltpu.sync_copy(rows, out_hbm.at[pl.ds(tb + j * SUB, SUB)])


@functools.cache
def _sc_kernels():
    mesh = plsc.VectorSubcoreMesh(core_axis_name="c", subcore_axis_name="s")
    scatter = functools.partial(
        pl.kernel, mesh=mesh,
        out_type=[jax.ShapeDtypeStruct((P, D), jnp.float32),
                  jax.ShapeDtypeStruct((N,), jnp.int32),
                  jax.ShapeDtypeStruct((N,), jnp.int32),
                  jax.ShapeDtypeStruct((P,), jnp.float32)],
        scratch_types=[pltpu.VMEM((16,), jnp.int32),
                       pltpu.VMEM((CHUNK,), jnp.int32),
                       pltpu.VMEM((CHUNK,), jnp.int32),
                       pltpu.VMEM((CHUNK,), jnp.int32),
                       pltpu.VMEM((CHUNK,), jnp.int32),
                       pltpu.VMEM((SUB,), jnp.float32),
                       pltpu.VMEM((SUB,), jnp.int32),
                       pltpu.VMEM((SUB,), jnp.int32),
                       pltpu.VMEM((SUB, D), jnp.float32)],
    )(_sc_scatter_body)
    gather = functools.partial(
        pl.kernel, mesh=mesh,
        out_type=jax.ShapeDtypeStruct((N, D), jnp.float32),
        scratch_types=[pltpu.VMEM((CHUNK,), jnp.int32),
                       pltpu.VMEM((CHUNK,), jnp.int32),
                       pltpu.VMEM((SUB,), jnp.int32),
                       pltpu.VMEM((SUB,), jnp.int32),
                       pltpu.VMEM((SUB, D), jnp.float32)],
    )(_sc_gather_body)
    return scatter, gather


def kernel(x, router_w, w1, w2, w3):
    x2d = x.reshape(N, D)
    (e0b, e1b, w0b, w1b, r0b, r1b, cnt, ps) = _router(x2d, router_w)
    e0 = e0b.reshape(N)
    e1 = e1b.reshape(N)
    r0 = r0b.reshape(N)
    r1 = r1b.reshape(N)

    counts = cnt.reshape(E)
    padded = ((counts + (TM - 1)) // TM) * TM
    ends = jnp.cumsum(padded)
    base = (ends - padded).astype(jnp.int32)
    tile_start = jnp.arange(NT, dtype=jnp.int32) * TM
    te = jnp.minimum(
        jnp.sum((tile_start[:, None] >= ends[None, :]).astype(jnp.int32),
                axis=1), E - 1).astype(jnp.int32)

    nt_used = ((ends[E - 1] + (TM - 1)) // TM).reshape(1).astype(jnp.int32)

    sc_scatter, sc_gather = _sc_kernels()
    xg, s0, s1, ws = sc_scatter(x2d, e0, e1, r0, r1,
                                w0b.reshape(N), w1b.reshape(N), base)
    yg = _mm(te, nt_used, xg, w1, w2, w3, ws.reshape(P, 1))
    out = sc_gather(yg, s0, s1)

    fraction = ps.reshape(E) / jnp.float32(N)
    target = jnp.ones((E,), jnp.float32) / E
    aux = jnp.sum(fraction * jnp.log(target)) * E
    return out.reshape(B, T, D), aux
